# baseline (speedup 1.0000x reference)
import functools
from functools import partial

import jax
import jax.numpy as jnp
from jax import lax
from jax.experimental import pallas as pl
from jax.experimental.pallas import tpu as pltpu

EPS = 1e-5
H = 64
HH = 32


def _node_init_body(e_ref, p_ref, wE_ref, wP_ref, b_ref, o_ref):
    e = e_ref[:, 0]
    onehot = (e[:, None] == lax.broadcasted_iota(jnp.int32, (1, 128), 1)).astype(jnp.float32)
    acc = jnp.dot(onehot, wE_ref[...], preferred_element_type=jnp.float32)
    acc += jnp.dot(p_ref[...], wP_ref[...], preferred_element_type=jnp.float32)
    o_ref[...] = acc + b_ref[...]


def _node_init(x_element, x_props, wE, wP, b, blk):
    n = x_element.shape[0]
    grid = n // blk
    return pl.pallas_call(
        _node_init_body,
        grid=(grid,),
        in_specs=[
            pl.BlockSpec((blk, 1), lambda i: (i, 0)),
            pl.BlockSpec((blk, 8), lambda i: (i, 0)),
            pl.BlockSpec((128, H), lambda i: (0, 0)),
            pl.BlockSpec((8, H), lambda i: (0, 0)),
            pl.BlockSpec((1, H), lambda i: (0, 0)),
        ],
        out_specs=pl.BlockSpec((blk, H), lambda i: (i, 0)),
        out_shape=jax.ShapeDtypeStruct((n, H), jnp.float32),
    )(x_element.astype(jnp.int32)[:, None], x_props, wE, wP, b[None])


def _rows_mm_body(x_ref, w_ref, b_ref, o_ref):
    o_ref[...] = jnp.dot(x_ref[...], w_ref[...], preferred_element_type=jnp.float32) + b_ref[...]


def _rows_mm(x, w, b, blk):
    n, k = x.shape
    return pl.pallas_call(
        _rows_mm_body,
        grid=(n // blk,),
        in_specs=[
            pl.BlockSpec((blk, k), lambda i: (i, 0)),
            pl.BlockSpec((k, w.shape[1]), lambda i: (0, 0)),
            pl.BlockSpec((1, w.shape[1]), lambda i: (0, 0)),
        ],
        out_specs=pl.BlockSpec((blk, w.shape[1]), lambda i: (i, 0)),
        out_shape=jax.ShapeDtypeStruct((n, w.shape[1]), jnp.float32),
    )(x, w, b[None])


def _proj_body(u_ref, wa_ref, ba_ref, wb_ref, oa0, oa1, ob0, ob1):
    u = u_ref[...]
    xa = jnp.dot(u, wa_ref[...], preferred_element_type=jnp.float32) + ba_ref[...]
    xb = jnp.dot(u, wb_ref[...], preferred_element_type=jnp.float32)
    oa0[...] = xa[:, :HH]
    oa1[...] = xa[:, HH:]
    ob0[...] = xb[:, :HH]
    ob1[...] = xb[:, HH:]


def _proj(u, wa, ba, wb, blk):
    n = u.shape[0]
    half = jax.ShapeDtypeStruct((n, HH), jnp.float32)
    return pl.pallas_call(
        _proj_body,
        grid=(n // blk,),
        in_specs=[
            pl.BlockSpec((blk, H), lambda i: (i, 0)),
            pl.BlockSpec((H, H), lambda i: (0, 0)),
            pl.BlockSpec((1, H), lambda i: (0, 0)),
            pl.BlockSpec((H, H), lambda i: (0, 0)),
        ],
        out_specs=[pl.BlockSpec((blk, HH), lambda i: (i, 0))] * 4,
        out_shape=[half] * 4,
    )(u, wa, ba[None], wb)


def _phaseA_body(x_ref, s_ref, c_ref, A_ref, B_ref, M_ref, m_ref, b_ref,
                 z_ref, ps_ref):
    cnt = c_ref[...]
    invc = 1.0 / jnp.maximum(cnt, 1.0)
    agg = jnp.dot(s_ref[...] * invc, M_ref[...], preferred_element_type=jnp.float32)
    agg += m_ref[...] * (cnt > 0.0).astype(jnp.float32)
    z = jnp.dot(x_ref[...], A_ref[...], preferred_element_type=jnp.float32)
    z += jnp.dot(agg, B_ref[...], preferred_element_type=jnp.float32)
    z += b_ref[...]
    z_ref[...] = z
    ps = jnp.concatenate(
        [jnp.sum(z, axis=0)[None], jnp.sum(z * z, axis=0)[None],
         jnp.zeros((6, H), jnp.float32)], axis=0)
    ps_ref[...] = ps[None]


def _phaseA(x, s, cnt, A, B, M, m, b, blk):
    n = x.shape[0]
    grid = n // blk
    return pl.pallas_call(
        _phaseA_body,
        grid=(grid,),
        in_specs=[
            pl.BlockSpec((blk, H), lambda i: (i, 0)),
            pl.BlockSpec((blk, H), lambda i: (i, 0)),
            pl.BlockSpec((blk, 1), lambda i: (i, 0)),
            pl.BlockSpec((H, H), lambda i: (0, 0)),
            pl.BlockSpec((H, H), lambda i: (0, 0)),
            pl.BlockSpec((H, H), lambda i: (0, 0)),
            pl.BlockSpec((1, H), lambda i: (0, 0)),
            pl.BlockSpec((1, H), lambda i: (0, 0)),
        ],
        out_specs=[
            pl.BlockSpec((blk, H), lambda i: (i, 0)),
            pl.BlockSpec((1, 8, H), lambda i: (i, 0, 0)),
        ],
        out_shape=[
            jax.ShapeDtypeStruct((n, H), jnp.float32),
            jax.ShapeDtypeStruct((grid, 8, H), jnp.float32),
        ],
    )(x, s, cnt[:, None], A, B, M, m[None], b[None])


def _stats_body(count, ps_ref, g_ref, B_ref, o_ref):
    sums = jnp.sum(ps_ref[:, 0, :], axis=0)
    sqs = jnp.sum(ps_ref[:, 1, :], axis=0)
    mean = sums / count
    var = sqs / count - mean * mean
    a = g_ref[0] * lax.rsqrt(var + EPS)
    c = B_ref[0] - mean * a
    o_ref[...] = jnp.concatenate(
        [a[None], c[None], jnp.zeros((6, H), jnp.float32)], axis=0)


def _stats(ps, g, B, count):
    k = ps.shape[0]
    return pl.pallas_call(
        partial(_stats_body, float(count)),
        in_specs=[
            pl.BlockSpec((k, 8, H), lambda: (0, 0, 0)),
            pl.BlockSpec((1, H), lambda: (0, 0)),
            pl.BlockSpec((1, H), lambda: (0, 0)),
        ],
        out_specs=pl.BlockSpec((8, H), lambda: (0, 0)),
        out_shape=jax.ShapeDtypeStruct((8, H), jnp.float32),
    )(ps, g[None], B[None])


def _silu(x):
    return x * (1.0 / (1.0 + jnp.exp(-x)))


def _phaseB_body(has_res, z_ref, st_ref, *rest):
    z = z_ref[...]
    a = st_ref[0:1, :]
    c = st_ref[1:2, :]
    u = _silu(z * a + c)
    if has_res:
        u = u + rest[0][...]
        rest = rest[1:]
    rest[0][...] = u


def _phaseB(z, st, res, blk):
    n = z.shape[0]
    has_res = res is not None
    ins = [z, st] + ([res] if has_res else [])
    specs = [
        pl.BlockSpec((blk, H), lambda i: (i, 0)),
        pl.BlockSpec((8, H), lambda i: (0, 0)),
    ] + ([pl.BlockSpec((blk, H), lambda i: (i, 0))] if has_res else [])
    return pl.pallas_call(
        partial(_phaseB_body, has_res),
        grid=(n // blk,),
        in_specs=specs,
        out_specs=pl.BlockSpec((blk, H), lambda i: (i, 0)),
        out_shape=jax.ShapeDtypeStruct((n, H), jnp.float32),
    )(*ins)


def _final_body(ae_ref, le_ref, ca_ref, cl_ref, wa_ref, wb_ref, b_ref, o_ref):
    ae = ae_ref[...] * (1.0 / jnp.maximum(ca_ref[...], 1.0))
    le = le_ref[...] * (1.0 / jnp.maximum(cl_ref[...], 1.0))
    g = jnp.dot(ae, wa_ref[...], preferred_element_type=jnp.float32)
    g += jnp.dot(le, wb_ref[...], preferred_element_type=jnp.float32)
    g += b_ref[...]
    o_ref[...] = _silu(g)


def _final(aS, lS, ca, cl, wa, wb, b, ng):
    return pl.pallas_call(
        _final_body,
        in_specs=[
            pl.BlockSpec((ng, H), lambda: (0, 0)),
            pl.BlockSpec((ng, H), lambda: (0, 0)),
            pl.BlockSpec((ng, 1), lambda: (0, 0)),
            pl.BlockSpec((ng, 1), lambda: (0, 0)),
            pl.BlockSpec((H, H), lambda: (0, 0)),
            pl.BlockSpec((H, H), lambda: (0, 0)),
            pl.BlockSpec((1, H), lambda: (0, 0)),
        ],
        out_specs=pl.BlockSpec((ng, H), lambda: (0, 0)),
        out_shape=jax.ShapeDtypeStruct((ng, H), jnp.float32),
    )(aS, lS, ca[:, None], cl[:, None], wa, wb, b[None])


def _edge_pass1(xa0, xa1, xb0, xb1, eterm0, eterm1, src, dst):
    h0 = xa0[dst] + xb0[src] + eterm0
    h1 = xa1[dst] + xb1[src] + eterm1
    h = jnp.concatenate([h0, h1], axis=1)
    ps = jnp.zeros((1, 8, H), jnp.float32)
    ps = ps.at[0, 0].set(jnp.sum(h, axis=0))
    ps = ps.at[0, 1].set(jnp.sum(h * h, axis=0))
    return h0, h1, ps


def _edge_pass2(h0, h1, st, idx, nseg):
    a = st[0]
    c = st[1]
    h = jnp.concatenate([h0, h1], axis=1)
    s = _silu(h * a + c)
    return jax.ops.segment_sum(s, idx, num_segments=nseg)


def _seg_sum_rows(vals, idx, nseg):
    return jax.ops.segment_sum(vals, idx, num_segments=nseg)


def _seg_counts(idx, nseg):
    return jax.ops.segment_sum(jnp.ones(idx.shape, jnp.float32), idx, num_segments=nseg)


def _conv(p, x, src, dst, eterm0, eterm1, cnt, n, blk):
    xa0, xa1, xb0, xb1 = _proj(x, p['mW1'][:H], p['mb1'], p['mW1'][H:2 * H], blk)
    h0, h1, ps = _edge_pass1(xa0, xa1, xb0, xb1, eterm0, eterm1, src, dst)
    st = _stats(ps, p['mg'], p['mB'], src.shape[0])
    S = _edge_pass2(h0, h1, st, dst, n)
    z, psz = _phaseA(x, S, cnt, p['uW'][:H], p['uW'][H:], p['mW2'], p['mb2'],
                     p['ub'], blk)
    stz = _stats(psz, p['ug'], p['uB'], n)
    return _phaseB(z, stz, x, blk)


_EYE = None


def kernel(x_element, x_props, edge_index, edge_attr, batch, line_graph_x,
           line_graph_edge_index, line_graph_edge_attr, line_graph_batch_mapping,
           params):
    n = x_element.shape[0]
    ne = edge_attr.shape[0]
    ng = 256
    blk_n = 2000
    blk_e = 4000

    src = edge_index[0].astype(jnp.int32)
    dst = edge_index[1].astype(jnp.int32)
    lsrc = line_graph_edge_index[0].astype(jnp.int32)
    ldst = line_graph_edge_index[1].astype(jnp.int32)
    batch32 = batch.astype(jnp.int32)
    lgbm = line_graph_batch_mapping.astype(jnp.int32)

    cnt_dst = _seg_counts(dst, n)
    cnt_ldst = _seg_counts(ldst, ne)
    cnt_src = _seg_counts(src, n)
    cnt_batch = _seg_counts(batch32, ng)
    line_batch = batch32[lgbm]
    cnt_lbatch = _seg_counts(line_batch, ng)

    emb_pad = jnp.zeros((128, 32), jnp.float32).at[:100].set(params['emb'])
    wE = emb_pad @ params['node_W'][:32]
    x = _node_init(x_element, x_props, wE, params['node_W'][32:],
                   params['node_b'], blk_n)
    lx = _rows_mm(line_graph_x, params['line_W'], params['line_b'], blk_e)

    def eterms(p, ea, blk):
        t = _rows_mm(ea, p['mW1'][2 * H:], jnp.zeros((H,), jnp.float32), blk)
        return t[:, :HH], t[:, HH:]

    eye = jnp.eye(H, dtype=jnp.float32)
    zH = jnp.zeros((H,), jnp.float32)

    for i in range(3):
        p = params['atom'][i]
        et0, et1 = eterms(p, edge_attr, blk_e)
        x = _conv(p, x, src, dst, et0, et1, cnt_dst, n, blk_n)
        q = params['line'][i]
        lt0, lt1 = eterms(q, line_graph_edge_attr, blk_e)
        lx = _conv(q, lx, lsrc, ldst, lt0, lt1, cnt_ldst, ne, blk_e)
        Sa = _seg_sum_rows(lx, src, n)
        qb = params['b2a'][i]
        z, psz = _phaseA(x, Sa, cnt_src, qb['W'][:H], qb['W'][H:], eye, zH,
                         qb['b'], blk_n)
        stz = _stats(psz, qb['g'], qb['B'], n)
        x = _phaseB(z, stz, None, blk_n)
    for i in range(3, 5):
        p = params['atom'][i]
        et0, et1 = eterms(p, edge_attr, blk_e)
        x = _conv(p, x, src, dst, et0, et1, cnt_dst, n, blk_n)

    aS = _seg_sum_rows(x, batch32, ng)
    lS = _seg_sum_rows(lx, line_batch, ng)
    return _final(aS, lS, cnt_batch, cnt_lbatch,
                  params['out_W'][:H], params['out_W'][H:], params['out_b'], ng)


# speedup vs baseline: 1.9727x; 1.9727x over previous
import functools
from functools import partial

import jax
import jax.numpy as jnp
from jax import lax
from jax.experimental import pallas as pl
from jax.experimental.pallas import tpu as pltpu
from jax.experimental.pallas import tpu_sc as plsc

EPS = 1e-5
H = 64
HH = 32


def _node_init_body(e_ref, p_ref, wE_ref, wP_ref, b_ref, o_ref):
    e = e_ref[:, 0]
    onehot = (e[:, None] == lax.broadcasted_iota(jnp.int32, (1, 128), 1)).astype(jnp.float32)
    acc = jnp.dot(onehot, wE_ref[...], preferred_element_type=jnp.float32)
    acc += jnp.dot(p_ref[...], wP_ref[...], preferred_element_type=jnp.float32)
    o_ref[...] = acc + b_ref[...]


def _node_init(x_element, x_props, wE, wP, b, blk):
    n = x_element.shape[0]
    grid = n // blk
    return pl.pallas_call(
        _node_init_body,
        grid=(grid,),
        in_specs=[
            pl.BlockSpec((blk, 1), lambda i: (i, 0)),
            pl.BlockSpec((blk, 8), lambda i: (i, 0)),
            pl.BlockSpec((128, H), lambda i: (0, 0)),
            pl.BlockSpec((8, H), lambda i: (0, 0)),
            pl.BlockSpec((1, H), lambda i: (0, 0)),
        ],
        out_specs=pl.BlockSpec((blk, H), lambda i: (i, 0)),
        out_shape=jax.ShapeDtypeStruct((n, H), jnp.float32),
    )(x_element.astype(jnp.int32)[:, None], x_props, wE, wP, b[None])


def _rows_mm_body(x_ref, w_ref, b_ref, o_ref):
    o_ref[...] = jnp.dot(x_ref[...], w_ref[...], preferred_element_type=jnp.float32) + b_ref[...]


def _rows_mm(x, w, b, blk):
    n, k = x.shape
    return pl.pallas_call(
        _rows_mm_body,
        grid=(n // blk,),
        in_specs=[
            pl.BlockSpec((blk, k), lambda i: (i, 0)),
            pl.BlockSpec((k, w.shape[1]), lambda i: (0, 0)),
            pl.BlockSpec((1, w.shape[1]), lambda i: (0, 0)),
        ],
        out_specs=pl.BlockSpec((blk, w.shape[1]), lambda i: (i, 0)),
        out_shape=jax.ShapeDtypeStruct((n, w.shape[1]), jnp.float32),
    )(x, w, b[None])


def _proj_body(u_ref, wa_ref, ba_ref, wb_ref, oa, ob):
    u = u_ref[...]
    oa[...] = jnp.dot(u, wa_ref[...], preferred_element_type=jnp.float32) + ba_ref[...]
    ob[...] = jnp.dot(u, wb_ref[...], preferred_element_type=jnp.float32)


def _proj(u, wa, ba, wb, blk):
    n = u.shape[0]
    full = jax.ShapeDtypeStruct((n, H), jnp.float32)
    return pl.pallas_call(
        _proj_body,
        grid=(n // blk,),
        in_specs=[
            pl.BlockSpec((blk, H), lambda i: (i, 0)),
            pl.BlockSpec((H, H), lambda i: (0, 0)),
            pl.BlockSpec((1, H), lambda i: (0, 0)),
            pl.BlockSpec((H, H), lambda i: (0, 0)),
        ],
        out_specs=[pl.BlockSpec((blk, H), lambda i: (i, 0))] * 2,
        out_shape=[full] * 2,
    )(u, wa, ba[None], wb)


def _phaseA_body(x_ref, s_ref, c_ref, A_ref, B_ref, M_ref, m_ref, b_ref,
                 z_ref, ps_ref):
    cnt = c_ref[...]
    invc = 1.0 / jnp.maximum(cnt, 1.0)
    agg = jnp.dot(s_ref[...] * invc, M_ref[...], preferred_element_type=jnp.float32)
    agg += m_ref[...] * (cnt > 0.0).astype(jnp.float32)
    z = jnp.dot(x_ref[...], A_ref[...], preferred_element_type=jnp.float32)
    z += jnp.dot(agg, B_ref[...], preferred_element_type=jnp.float32)
    z += b_ref[...]
    z_ref[...] = z
    ps = jnp.concatenate(
        [jnp.sum(z, axis=0)[None], jnp.sum(z * z, axis=0)[None]], axis=0)
    ps_ref[...] = ps[None]


def _phaseA(x, s, cnt, A, B, M, m, b, blk):
    n = x.shape[0]
    grid = n // blk
    return pl.pallas_call(
        _phaseA_body,
        grid=(grid,),
        in_specs=[
            pl.BlockSpec((blk, H), lambda i: (i, 0)),
            pl.BlockSpec((blk, H), lambda i: (i, 0)),
            pl.BlockSpec((blk, 1), lambda i: (i, 0)),
            pl.BlockSpec((H, H), lambda i: (0, 0)),
            pl.BlockSpec((H, H), lambda i: (0, 0)),
            pl.BlockSpec((H, H), lambda i: (0, 0)),
            pl.BlockSpec((1, H), lambda i: (0, 0)),
            pl.BlockSpec((1, H), lambda i: (0, 0)),
        ],
        out_specs=[
            pl.BlockSpec((blk, H), lambda i: (i, 0)),
            pl.BlockSpec((1, 2, H), lambda i: (i, 0, 0)),
        ],
        out_shape=[
            jax.ShapeDtypeStruct((n, H), jnp.float32),
            jax.ShapeDtypeStruct((grid, 2, H), jnp.float32),
        ],
    )(x, s, cnt[:, None], A, B, M, m[None], b[None])


def _stats_body(count, ps_ref, g_ref, B_ref, o_ref):
    sums = jnp.sum(ps_ref[:, 0, :], axis=0)
    sqs = jnp.sum(ps_ref[:, 1, :], axis=0)
    mean = sums / count
    var = sqs / count - mean * mean
    a = g_ref[0] * lax.rsqrt(var + EPS)
    c = B_ref[0] - mean * a
    o_ref[...] = jnp.concatenate([a[None], c[None]], axis=0)


def _stats(ps, g, B, count):
    k = ps.shape[0]
    return pl.pallas_call(
        partial(_stats_body, float(count)),
        in_specs=[
            pl.BlockSpec((k, 2, H), lambda: (0, 0, 0)),
            pl.BlockSpec((1, H), lambda: (0, 0)),
            pl.BlockSpec((1, H), lambda: (0, 0)),
        ],
        out_specs=pl.BlockSpec((2, H), lambda: (0, 0)),
        out_shape=jax.ShapeDtypeStruct((2, H), jnp.float32),
    )(ps, g[None], B[None])


def _silu(x):
    return x * (1.0 / (1.0 + jnp.exp(-x)))


def _phaseB_body(has_res, z_ref, st_ref, *rest):
    z = z_ref[...]
    a = st_ref[0:1, :]
    c = st_ref[1:2, :]
    u = _silu(z * a + c)
    if has_res:
        u = u + rest[0][...]
        rest = rest[1:]
    rest[0][...] = u


def _phaseB(z, st, res, blk):
    n = z.shape[0]
    has_res = res is not None
    ins = [z, st] + ([res] if has_res else [])
    specs = [
        pl.BlockSpec((blk, H), lambda i: (i, 0)),
        pl.BlockSpec((2, H), lambda i: (0, 0)),
    ] + ([pl.BlockSpec((blk, H), lambda i: (i, 0))] if has_res else [])
    return pl.pallas_call(
        partial(_phaseB_body, has_res),
        grid=(n // blk,),
        in_specs=specs,
        out_specs=pl.BlockSpec((blk, H), lambda i: (i, 0)),
        out_shape=jax.ShapeDtypeStruct((n, H), jnp.float32),
    )(*ins)


def _final_body(ae_ref, le_ref, ca_ref, cl_ref, wa_ref, wb_ref, b_ref, o_ref):
    ae = ae_ref[...] * (1.0 / jnp.maximum(ca_ref[...], 1.0))
    le = le_ref[...] * (1.0 / jnp.maximum(cl_ref[...], 1.0))
    g = jnp.dot(ae, wa_ref[...], preferred_element_type=jnp.float32)
    g += jnp.dot(le, wb_ref[...], preferred_element_type=jnp.float32)
    g += b_ref[...]
    o_ref[...] = _silu(g)


def _final(aS, lS, ca, cl, wa, wb, b, ng):
    return pl.pallas_call(
        _final_body,
        in_specs=[
            pl.BlockSpec((ng, H), lambda: (0, 0)),
            pl.BlockSpec((ng, H), lambda: (0, 0)),
            pl.BlockSpec((ng, 1), lambda: (0, 0)),
            pl.BlockSpec((ng, 1), lambda: (0, 0)),
            pl.BlockSpec((H, H), lambda: (0, 0)),
            pl.BlockSpec((H, H), lambda: (0, 0)),
            pl.BlockSpec((1, H), lambda: (0, 0)),
        ],
        out_specs=pl.BlockSpec((ng, H), lambda: (0, 0)),
        out_shape=jax.ShapeDtypeStruct((ng, H), jnp.float32),
    )(aS, lS, ca[:, None], cl[:, None], wa, wb, b[None])


_NC = 2
_NS = 16
_NW = _NC * _NS
_L = 16


def _sc_mesh():
    return plsc.VectorSubcoreMesh(core_axis_name="c", subcore_axis_name="s",
                                  num_cores=_NC, num_subcores=_NS)


def _wid():
    return lax.axis_index("c") * _NS + lax.axis_index("s")


def _sc_pass1_make(n, e, c1):
    per = e // _NW
    nt = per // c1
    assert per * _NW == e and nt * c1 == per and c1 % 8 == 0

    @functools.partial(
        pl.kernel,
        out_type=[
            jax.ShapeDtypeStruct((2, e, HH), jnp.float32),
            jax.ShapeDtypeStruct((_NW, 2, H), jnp.float32),
        ],
        mesh=_sc_mesh(),
        compiler_params=pltpu.CompilerParams(use_tc_tiling_on_sc=False),
        scratch_types=[
            pltpu.VMEM((c1,), jnp.int32),
            pltpu.VMEM((c1,), jnp.int32),
            pltpu.VMEM((c1, H), jnp.float32),
            pltpu.VMEM((c1, H), jnp.float32),
            pltpu.VMEM((c1, H), jnp.float32),
            pltpu.VMEM((c1, HH), jnp.float32),
            pltpu.VMEM((c1, HH), jnp.float32),
            pltpu.VMEM((2, H), jnp.float32),
            pltpu.SemaphoreType.DMA,
            pltpu.SemaphoreType.DMA,
        ],
    )
    def k(xa_hbm, xb_hbm, et_hbm, src_hbm, dst_hbm, h_hbm, part_hbm,
          idxd_v, idxs_v, a_v, b_v, e_v, o0_v, o1_v, p_v, sem1, sem2):
        w = _wid()
        zero = jnp.zeros((_L,), jnp.float32)

        def outer(t, acc):
            base = (w * per + t * c1).astype(jnp.int32)
            pltpu.sync_copy(dst_hbm.at[pl.ds(base, c1)], idxd_v)
            pltpu.sync_copy(src_hbm.at[pl.ds(base, c1)], idxs_v)
            cpa = pltpu.async_copy(xa_hbm.at[idxd_v], a_v, sem1)
            cpb = pltpu.async_copy(xb_hbm.at[idxs_v], b_v, sem2)
            pltpu.sync_copy(et_hbm.at[pl.ds(base, c1)], e_v)
            cpa.wait()
            cpb.wait()

            def inner(r, acc):
                s10, s11, s12, s13, s20, s21, s22, s23 = acc
                h0 = (a_v[r, pl.ds(0, _L)] + b_v[r, pl.ds(0, _L)]
                      + e_v[r, pl.ds(0, _L)])
                h1 = (a_v[r, pl.ds(_L, _L)] + b_v[r, pl.ds(_L, _L)]
                      + e_v[r, pl.ds(_L, _L)])
                h2 = (a_v[r, pl.ds(2 * _L, _L)] + b_v[r, pl.ds(2 * _L, _L)]
                      + e_v[r, pl.ds(2 * _L, _L)])
                h3 = (a_v[r, pl.ds(3 * _L, _L)] + b_v[r, pl.ds(3 * _L, _L)]
                      + e_v[r, pl.ds(3 * _L, _L)])
                o0_v[r, pl.ds(0, _L)] = h0
                o0_v[r, pl.ds(_L, _L)] = h1
                o1_v[r, pl.ds(0, _L)] = h2
                o1_v[r, pl.ds(_L, _L)] = h3
                return (s10 + h0, s11 + h1, s12 + h2, s13 + h3,
                        s20 + h0 * h0, s21 + h1 * h1, s22 + h2 * h2,
                        s23 + h3 * h3)

            acc = lax.fori_loop(0, c1, inner, acc)
            pltpu.sync_copy(o0_v, h_hbm.at[0, pl.ds(base, c1)])
            pltpu.sync_copy(o1_v, h_hbm.at[1, pl.ds(base, c1)])
            return acc

        acc = lax.fori_loop(0, nt, outer, (zero,) * 8)
        p_v[0, pl.ds(0, _L)] = acc[0]
        p_v[0, pl.ds(_L, _L)] = acc[1]
        p_v[0, pl.ds(2 * _L, _L)] = acc[2]
        p_v[0, pl.ds(3 * _L, _L)] = acc[3]
        p_v[1, pl.ds(0, _L)] = acc[4]
        p_v[1, pl.ds(_L, _L)] = acc[5]
        p_v[1, pl.ds(2 * _L, _L)] = acc[6]
        p_v[1, pl.ds(3 * _L, _L)] = acc[7]
        pltpu.sync_copy(p_v, part_hbm.at[w])

    return k


@functools.lru_cache(maxsize=None)
def _sc_pass1_cached(n, e, c1):
    return _sc_pass1_make(n, e, c1)


def _edge_pass2(h0, h1, st, idx, nseg):
    a = st[0]
    c = st[1]
    h = jnp.concatenate([h0, h1], axis=1)
    s = _silu(h * a + c)
    return jax.ops.segment_sum(s, idx, num_segments=nseg)


def _seg_sum_rows(vals, idx, nseg):
    return jax.ops.segment_sum(vals, idx, num_segments=nseg)


def _seg_counts(idx, nseg):
    return jax.ops.segment_sum(jnp.ones(idx.shape, jnp.float32), idx, num_segments=nseg)


def _conv(p, x, src, dst, eterm, cnt, n, blk):
    xa, xb = _proj(x, p['mW1'][:H], p['mb1'], p['mW1'][H:2 * H], blk)
    e = src.shape[0]
    h, ps = _sc_pass1_cached(n, e, 200)(xa, xb, eterm, src, dst)
    st = _stats(ps, p['mg'], p['mB'], e)
    S = _edge_pass2(h[0], h[1], st, dst, n)
    z, psz = _phaseA(x, S, cnt, p['uW'][:H], p['uW'][H:], p['mW2'], p['mb2'],
                     p['ub'], blk)
    stz = _stats(psz, p['ug'], p['uB'], n)
    return _phaseB(z, stz, x, blk)


_EYE = None


def kernel(x_element, x_props, edge_index, edge_attr, batch, line_graph_x,
           line_graph_edge_index, line_graph_edge_attr, line_graph_batch_mapping,
           params):
    n = x_element.shape[0]
    ne = edge_attr.shape[0]
    ng = 256
    blk_n = 2000
    blk_e = 4000

    src = edge_index[0].astype(jnp.int32)
    dst = edge_index[1].astype(jnp.int32)
    lsrc = line_graph_edge_index[0].astype(jnp.int32)
    ldst = line_graph_edge_index[1].astype(jnp.int32)
    batch32 = batch.astype(jnp.int32)
    lgbm = line_graph_batch_mapping.astype(jnp.int32)

    cnt_dst = _seg_counts(dst, n)
    cnt_ldst = _seg_counts(ldst, ne)
    cnt_src = _seg_counts(src, n)
    cnt_batch = _seg_counts(batch32, ng)
    line_batch = batch32[lgbm]
    cnt_lbatch = _seg_counts(line_batch, ng)

    emb_pad = jnp.zeros((128, 32), jnp.float32).at[:100].set(params['emb'])
    wE = emb_pad @ params['node_W'][:32]
    x = _node_init(x_element, x_props, wE, params['node_W'][32:],
                   params['node_b'], blk_n)
    lx = _rows_mm(line_graph_x, params['line_W'], params['line_b'], blk_e)

    def eterms(p, ea, blk):
        return _rows_mm(ea, p['mW1'][2 * H:], jnp.zeros((H,), jnp.float32), blk)

    eye = jnp.eye(H, dtype=jnp.float32)
    zH = jnp.zeros((H,), jnp.float32)

    for i in range(3):
        p = params['atom'][i]
        et = eterms(p, edge_attr, blk_e)
        x = _conv(p, x, src, dst, et, cnt_dst, n, blk_n)
        q = params['line'][i]
        lt = eterms(q, line_graph_edge_attr, blk_e)
        lx = _conv(q, lx, lsrc, ldst, lt, cnt_ldst, ne, blk_e)
        Sa = _seg_sum_rows(lx, src, n)
        qb = params['b2a'][i]
        z, psz = _phaseA(x, Sa, cnt_src, qb['W'][:H], qb['W'][H:], eye, zH,
                         qb['b'], blk_n)
        stz = _stats(psz, qb['g'], qb['B'], n)
        x = _phaseB(z, stz, None, blk_n)
    for i in range(3, 5):
        p = params['atom'][i]
        et = eterms(p, edge_attr, blk_e)
        x = _conv(p, x, src, dst, et, cnt_dst, n, blk_n)

    aS = _seg_sum_rows(x, batch32, ng)
    lS = _seg_sum_rows(lx, line_batch, ng)
    return _final(aS, lS, cnt_batch, cnt_lbatch,
                  params['out_W'][:H], params['out_W'][H:], params['out_b'], ng)


# speedup vs baseline: 1.9896x; 1.0085x over previous
import functools
from functools import partial

import jax
import jax.numpy as jnp
from jax import lax
from jax.experimental import pallas as pl
from jax.experimental.pallas import tpu as pltpu
from jax.experimental.pallas import tpu_sc as plsc

EPS = 1e-5
H = 64
HH = 32


def _node_init_body(e_ref, p_ref, wE_ref, wP_ref, b_ref, o_ref):
    e = e_ref[:, 0]
    onehot = (e[:, None] == lax.broadcasted_iota(jnp.int32, (1, 128), 1)).astype(jnp.float32)
    acc = jnp.dot(onehot, wE_ref[...], preferred_element_type=jnp.float32)
    acc += jnp.dot(p_ref[...], wP_ref[...], preferred_element_type=jnp.float32)
    o_ref[...] = acc + b_ref[...]


def _node_init(x_element, x_props, wE, wP, b, blk):
    n = x_element.shape[0]
    grid = n // blk
    return pl.pallas_call(
        _node_init_body,
        grid=(grid,),
        in_specs=[
            pl.BlockSpec((blk, 1), lambda i: (i, 0)),
            pl.BlockSpec((blk, 8), lambda i: (i, 0)),
            pl.BlockSpec((128, H), lambda i: (0, 0)),
            pl.BlockSpec((8, H), lambda i: (0, 0)),
            pl.BlockSpec((1, H), lambda i: (0, 0)),
        ],
        out_specs=pl.BlockSpec((blk, H), lambda i: (i, 0)),
        out_shape=jax.ShapeDtypeStruct((n, H), jnp.float32),
    )(x_element.astype(jnp.int32)[:, None], x_props, wE, wP, b[None])


def _rows_mm_body(x_ref, w_ref, b_ref, o_ref):
    o_ref[...] = jnp.dot(x_ref[...], w_ref[...], preferred_element_type=jnp.float32) + b_ref[...]


def _rows_mm(x, w, b, blk):
    n, k = x.shape
    return pl.pallas_call(
        _rows_mm_body,
        grid=(n // blk,),
        in_specs=[
            pl.BlockSpec((blk, k), lambda i: (i, 0)),
            pl.BlockSpec((k, w.shape[1]), lambda i: (0, 0)),
            pl.BlockSpec((1, w.shape[1]), lambda i: (0, 0)),
        ],
        out_specs=pl.BlockSpec((blk, w.shape[1]), lambda i: (i, 0)),
        out_shape=jax.ShapeDtypeStruct((n, w.shape[1]), jnp.float32),
    )(x, w, b[None])


def _proj_body(u_ref, wa_ref, ba_ref, wb_ref, oa, ob):
    u = u_ref[...]
    oa[...] = jnp.dot(u, wa_ref[...], preferred_element_type=jnp.float32) + ba_ref[...]
    ob[...] = jnp.dot(u, wb_ref[...], preferred_element_type=jnp.float32)


def _proj(u, wa, ba, wb, blk):
    n = u.shape[0]
    full = jax.ShapeDtypeStruct((n, H), jnp.float32)
    return pl.pallas_call(
        _proj_body,
        grid=(n // blk,),
        in_specs=[
            pl.BlockSpec((blk, H), lambda i: (i, 0)),
            pl.BlockSpec((H, H), lambda i: (0, 0)),
            pl.BlockSpec((1, H), lambda i: (0, 0)),
            pl.BlockSpec((H, H), lambda i: (0, 0)),
        ],
        out_specs=[pl.BlockSpec((blk, H), lambda i: (i, 0))] * 2,
        out_shape=[full] * 2,
    )(u, wa, ba[None], wb)


def _phaseA_body(x_ref, s0_ref, s1_ref, c_ref, A_ref, B_ref, M_ref, m_ref,
                 b_ref, z_ref, ps_ref):
    cnt = c_ref[...]
    invc = 1.0 / jnp.maximum(cnt, 1.0)
    s = jnp.concatenate([s0_ref[0], s1_ref[0]], axis=1)
    agg = jnp.dot(s * invc, M_ref[...], preferred_element_type=jnp.float32)
    agg += m_ref[...] * (cnt > 0.0).astype(jnp.float32)
    z = jnp.dot(x_ref[...], A_ref[...], preferred_element_type=jnp.float32)
    z += jnp.dot(agg, B_ref[...], preferred_element_type=jnp.float32)
    z += b_ref[...]
    z_ref[...] = z
    ps = jnp.concatenate(
        [jnp.sum(z, axis=0)[None], jnp.sum(z * z, axis=0)[None]], axis=0)
    ps_ref[...] = ps[None]


def _phaseA(x, s, cnt, A, B, M, m, b, blk):
    n = x.shape[0]
    grid = n // blk
    return pl.pallas_call(
        _phaseA_body,
        grid=(grid,),
        in_specs=[
            pl.BlockSpec((blk, H), lambda i: (i, 0)),
            pl.BlockSpec((1, blk, HH), lambda i: (0, i, 0)),
            pl.BlockSpec((1, blk, HH), lambda i: (1, i, 0)),
            pl.BlockSpec((blk, 1), lambda i: (i, 0)),
            pl.BlockSpec((H, H), lambda i: (0, 0)),
            pl.BlockSpec((H, H), lambda i: (0, 0)),
            pl.BlockSpec((H, H), lambda i: (0, 0)),
            pl.BlockSpec((1, H), lambda i: (0, 0)),
            pl.BlockSpec((1, H), lambda i: (0, 0)),
        ],
        out_specs=[
            pl.BlockSpec((blk, H), lambda i: (i, 0)),
            pl.BlockSpec((1, 2, H), lambda i: (i, 0, 0)),
        ],
        out_shape=[
            jax.ShapeDtypeStruct((n, H), jnp.float32),
            jax.ShapeDtypeStruct((grid, 2, H), jnp.float32),
        ],
    )(x, s, s, cnt[:, None], A, B, M, m[None], b[None])


def _stats_body(count, ps_ref, g_ref, B_ref, o_ref):
    sums = jnp.sum(ps_ref[:, 0, :], axis=0)
    sqs = jnp.sum(ps_ref[:, 1, :], axis=0)
    mean = sums / count
    var = sqs / count - mean * mean
    a = g_ref[0] * lax.rsqrt(var + EPS)
    c = B_ref[0] - mean * a
    o_ref[...] = jnp.concatenate([a[None], c[None]], axis=0)


def _stats(ps, g, B, count):
    k = ps.shape[0]
    return pl.pallas_call(
        partial(_stats_body, float(count)),
        grid=(1,),
        in_specs=[
            pl.BlockSpec((k, 2, H), lambda i: (0, 0, 0)),
            pl.BlockSpec((1, H), lambda i: (0, 0)),
            pl.BlockSpec((1, H), lambda i: (0, 0)),
        ],
        out_specs=pl.BlockSpec((2, H), lambda i: (0, 0)),
        out_shape=jax.ShapeDtypeStruct((2, H), jnp.float32),
    )(ps, g[None], B[None])


def _silu(x):
    return x * (1.0 / (1.0 + jnp.exp(-x)))


def _silu_h_body(h_ref, st_ref, o_ref):
    a0 = st_ref[0:1, :HH]
    a1 = st_ref[0:1, HH:]
    c0 = st_ref[1:2, :HH]
    c1 = st_ref[1:2, HH:]
    o_ref[0] = _silu(h_ref[0] * a0 + c0)
    o_ref[1] = _silu(h_ref[1] * a1 + c1)


def _silu_h(h, st, blk):
    e = h.shape[1]
    return pl.pallas_call(
        _silu_h_body,
        grid=(e // blk,),
        in_specs=[
            pl.BlockSpec((2, blk, HH), lambda i: (0, i, 0)),
            pl.BlockSpec((2, H), lambda i: (0, 0)),
        ],
        out_specs=pl.BlockSpec((2, blk, HH), lambda i: (0, i, 0)),
        out_shape=jax.ShapeDtypeStruct((2, e, HH), jnp.float32),
    )(h, st)


def _phaseB_body(has_res, stacked, z_ref, st_ref, *rest):
    z = z_ref[...]
    a = st_ref[0:1, :]
    c = st_ref[1:2, :]
    u = _silu(z * a + c)
    if has_res:
        u = u + rest[0][...]
        rest = rest[1:]
    rest[0][...] = u
    if stacked:
        rest[1][0] = u[:, :HH]
        rest[1][1] = u[:, HH:]


def _phaseB(z, st, res, blk, stacked=False):
    n = z.shape[0]
    has_res = res is not None
    ins = [z, st] + ([res] if has_res else [])
    specs = [
        pl.BlockSpec((blk, H), lambda i: (i, 0)),
        pl.BlockSpec((2, H), lambda i: (0, 0)),
    ] + ([pl.BlockSpec((blk, H), lambda i: (i, 0))] if has_res else [])
    out_specs = [pl.BlockSpec((blk, H), lambda i: (i, 0))]
    out_shape = [jax.ShapeDtypeStruct((n, H), jnp.float32)]
    if stacked:
        out_specs.append(pl.BlockSpec((2, blk, HH), lambda i: (0, i, 0)))
        out_shape.append(jax.ShapeDtypeStruct((2, n, HH), jnp.float32))
    out = pl.pallas_call(
        partial(_phaseB_body, has_res, stacked),
        grid=(n // blk,),
        in_specs=specs,
        out_specs=out_specs if stacked else out_specs[0],
        out_shape=out_shape if stacked else out_shape[0],
    )(*ins)
    return out


def _final_body(a0_ref, a1_ref, l0_ref, l1_ref, ca_ref, cl_ref, wa_ref,
                wb_ref, b_ref, o_ref):
    aS = jnp.concatenate([a0_ref[0], a1_ref[0]], axis=1)
    lS = jnp.concatenate([l0_ref[0], l1_ref[0]], axis=1)
    ae = aS * (1.0 / jnp.maximum(ca_ref[...], 1.0))
    le = lS * (1.0 / jnp.maximum(cl_ref[...], 1.0))
    g = jnp.dot(ae, wa_ref[...], preferred_element_type=jnp.float32)
    g += jnp.dot(le, wb_ref[...], preferred_element_type=jnp.float32)
    g += b_ref[...]
    o_ref[...] = _silu(g)


def _final(aS, lS, ca, cl, wa, wb, b, ng):
    stk = lambda j: pl.BlockSpec((1, ng, HH), lambda: (j, 0, 0))
    return pl.pallas_call(
        _final_body,
        grid=(1,),
        in_specs=[
            pl.BlockSpec((1, ng, HH), lambda i: (0, 0, 0)),
            pl.BlockSpec((1, ng, HH), lambda i: (1, 0, 0)),
            pl.BlockSpec((1, ng, HH), lambda i: (0, 0, 0)),
            pl.BlockSpec((1, ng, HH), lambda i: (1, 0, 0)),
            pl.BlockSpec((ng, 1), lambda i: (0, 0)),
            pl.BlockSpec((ng, 1), lambda i: (0, 0)),
            pl.BlockSpec((H, H), lambda i: (0, 0)),
            pl.BlockSpec((H, H), lambda i: (0, 0)),
            pl.BlockSpec((1, H), lambda i: (0, 0)),
        ],
        out_specs=pl.BlockSpec((ng, H), lambda i: (0, 0)),
        out_shape=jax.ShapeDtypeStruct((ng, H), jnp.float32),
    )(aS, aS, lS, lS, ca[:, None], cl[:, None], wa, wb, b[None])


_NC = 2
_NS = 16
_NW = _NC * _NS
_L = 16


def _sc_mesh():
    return plsc.VectorSubcoreMesh(core_axis_name="c", subcore_axis_name="s",
                                  num_cores=_NC, num_subcores=_NS)


def _wid():
    return lax.axis_index("c") * _NS + lax.axis_index("s")


def _sc_pass1_make(n, e, c1):
    per = e // _NW
    nt = per // c1
    assert per * _NW == e and nt * c1 == per and c1 % 8 == 0

    @functools.partial(
        pl.kernel,
        out_type=[
            jax.ShapeDtypeStruct((2, e, HH), jnp.float32),
            jax.ShapeDtypeStruct((_NW, 2, H), jnp.float32),
        ],
        mesh=_sc_mesh(),
        compiler_params=pltpu.CompilerParams(use_tc_tiling_on_sc=False),
        scratch_types=[
            pltpu.VMEM((c1,), jnp.int32),
            pltpu.VMEM((c1,), jnp.int32),
            pltpu.VMEM((c1, H), jnp.float32),
            pltpu.VMEM((c1, H), jnp.float32),
            pltpu.VMEM((c1, H), jnp.float32),
            pltpu.VMEM((c1, HH), jnp.float32),
            pltpu.VMEM((c1, HH), jnp.float32),
            pltpu.VMEM((2, H), jnp.float32),
            pltpu.SemaphoreType.DMA,
            pltpu.SemaphoreType.DMA,
        ],
    )
    def k(xa_hbm, xb_hbm, et_hbm, src_hbm, dst_hbm, h_hbm, part_hbm,
          idxd_v, idxs_v, a_v, b_v, e_v, o0_v, o1_v, p_v, sem1, sem2):
        w = _wid()
        zero = jnp.zeros((_L,), jnp.float32)

        def outer(t, acc):
            base = (w * per + t * c1).astype(jnp.int32)
            pltpu.sync_copy(dst_hbm.at[pl.ds(base, c1)], idxd_v)
            pltpu.sync_copy(src_hbm.at[pl.ds(base, c1)], idxs_v)
            cpa = pltpu.async_copy(xa_hbm.at[idxd_v], a_v, sem1)
            cpb = pltpu.async_copy(xb_hbm.at[idxs_v], b_v, sem2)
            pltpu.sync_copy(et_hbm.at[pl.ds(base, c1)], e_v)
            cpa.wait()
            cpb.wait()

            def inner(r, acc):
                s10, s11, s12, s13, s20, s21, s22, s23 = acc
                h0 = (a_v[r, pl.ds(0, _L)] + b_v[r, pl.ds(0, _L)]
                      + e_v[r, pl.ds(0, _L)])
                h1 = (a_v[r, pl.ds(_L, _L)] + b_v[r, pl.ds(_L, _L)]
                      + e_v[r, pl.ds(_L, _L)])
                h2 = (a_v[r, pl.ds(2 * _L, _L)] + b_v[r, pl.ds(2 * _L, _L)]
                      + e_v[r, pl.ds(2 * _L, _L)])
                h3 = (a_v[r, pl.ds(3 * _L, _L)] + b_v[r, pl.ds(3 * _L, _L)]
                      + e_v[r, pl.ds(3 * _L, _L)])
                o0_v[r, pl.ds(0, _L)] = h0
                o0_v[r, pl.ds(_L, _L)] = h1
                o1_v[r, pl.ds(0, _L)] = h2
                o1_v[r, pl.ds(_L, _L)] = h3
                return (s10 + h0, s11 + h1, s12 + h2, s13 + h3,
                        s20 + h0 * h0, s21 + h1 * h1, s22 + h2 * h2,
                        s23 + h3 * h3)

            acc = lax.fori_loop(0, c1, inner, acc)
            pltpu.sync_copy(o0_v, h_hbm.at[0, pl.ds(base, c1)])
            pltpu.sync_copy(o1_v, h_hbm.at[1, pl.ds(base, c1)])
            return acc

        acc = lax.fori_loop(0, nt, outer, (zero,) * 8)
        p_v[0, pl.ds(0, _L)] = acc[0]
        p_v[0, pl.ds(_L, _L)] = acc[1]
        p_v[0, pl.ds(2 * _L, _L)] = acc[2]
        p_v[0, pl.ds(3 * _L, _L)] = acc[3]
        p_v[1, pl.ds(0, _L)] = acc[4]
        p_v[1, pl.ds(_L, _L)] = acc[5]
        p_v[1, pl.ds(2 * _L, _L)] = acc[6]
        p_v[1, pl.ds(3 * _L, _L)] = acc[7]
        pltpu.sync_copy(p_v, part_hbm.at[w])

    return k


@functools.lru_cache(maxsize=None)
def _sc_pass1_cached(n, e, c1):
    return _sc_pass1_make(n, e, c1)


def _fill_vmem(ref, rows, val):
    v = jnp.full((_L,), val, jnp.float32)

    def body(r, _):
        ref[r, pl.ds(0, _L)] = v
        ref[r, pl.ds(_L, _L)] = v
        return 0

    lax.fori_loop(0, rows, body, 0)


def _sc_scatter_small_make(r, nseg, ones, table_n):
    c = 400
    nstream = r // c
    assert nstream * c == r and c % _L == 0
    nchunk = -(-nseg // _CH)
    rows = _CH if nchunk > 1 else nseg
    per_t = rows // _NS
    zc = 128 if per_t % 128 == 0 else per_t
    nz = per_t // zc
    assert nz * zc == per_t
    npad = nchunk * rows

    ins = []
    if not ones:
        ins.append(jax.ShapeDtypeStruct((2, r, HH), jnp.float32))
    ins.append(jax.ShapeDtypeStruct((r,), jnp.int32))
    if table_n:
        ins.append(jax.ShapeDtypeStruct((table_n,), jnp.int32))

    scratch = [
        pltpu.VMEM((c,), jnp.int32),
        pltpu.VMEM((c,), jnp.int32),
        pltpu.VMEM((c,), jnp.int32),
        pltpu.VMEM((c, HH), jnp.float32),
        pltpu.VMEM((zc, HH), jnp.float32),
        pltpu.VMEM_SHARED((rows + _NS, HH), jnp.float32),
        pltpu.SemaphoreType.DMA,
    ]

    @functools.partial(
        pl.kernel,
        out_type=jax.ShapeDtypeStruct((2, npad, HH), jnp.float32),
        mesh=_sc_mesh(),
        compiler_params=pltpu.CompilerParams(
            use_tc_tiling_on_sc=False, needs_layout_passes=False),
        scratch_types=scratch,
    )
    def k(*refs):
        pos = 0
        if not ones:
            vals_hbm = refs[pos]
            pos += 1
        idx_hbm = refs[pos]
        pos += 1
        if table_n:
            tab_hbm = refs[pos]
            pos += 1
        (out_hbm, idxr_v, idxm_v, idxo_v, vals_v, zb_v, acc,
         sem) = refs[pos:pos + 8]

        cid = lax.axis_index("c")
        sid = lax.axis_index("s")
        trash = rows + sid

        _fill_vmem(zb_v, zc, 0.0)
        if ones:
            _fill_vmem(vals_v, c, 1.0)
        nt = (nstream - 1 - sid) // _NS + 1

        def chunk(b, _):
            base_off = b * rows

            def zloop(j, _):
                pltpu.sync_copy(zb_v, acc.at[pl.ds(sid * per_t + j * zc, zc)])
                return 0

            lax.fori_loop(0, nz, zloop, 0)
            plsc.subcore_barrier()

            def main(t, _):
                base = (sid + t * _NS) * c
                pltpu.sync_copy(idx_hbm.at[pl.ds(base, c)], idxr_v)
                if table_n:
                    pltpu.async_copy(tab_hbm.at[idxr_v], idxm_v, sem).wait()
                    iv = idxm_v
                else:
                    iv = idxr_v

                def off_loop(j, _):
                    o = iv[pl.ds(j * _L, _L)] - base_off
                    valid = (o >= 0) & (o < rows)
                    idxo_v[pl.ds(j * _L, _L)] = jnp.where(valid, o, trash)
                    return 0

                lax.fori_loop(0, c // _L, off_loop, 0)
                if not ones:
                    pltpu.sync_copy(vals_hbm.at[cid, pl.ds(base, c)], vals_v)
                pltpu.sync_copy(vals_v, acc.at[idxo_v], add=True)
                return 0

            lax.fori_loop(0, nt, main, 0)
            plsc.subcore_barrier()

            def dump(j, _):
                sl = pl.ds(sid * per_t + j * zc, zc)
                pltpu.sync_copy(
                    acc.at[sl],
                    out_hbm.at[cid, pl.ds(base_off + sid * per_t + j * zc, zc)])
                return 0

            lax.fori_loop(0, nz, dump, 0)
            plsc.subcore_barrier()
            return 0

        lax.fori_loop(0, nchunk, chunk, 0)

    return k


@functools.lru_cache(maxsize=None)
def _sc_scatter_small_cached(r, nseg, ones, table_n):
    return _sc_scatter_small_make(r, nseg, ones, table_n)


_CH = 28672


def _rpad(nseg):
    return -(-nseg // _CH) * _CH


def _sc_scatter_big_make(r, nseg, ones):
    sc = 800
    nstream = r // sc
    assert nstream * sc == r
    cap = 8192
    g2 = 256
    per_t = _CH // _NS
    nzc = per_t // 128
    nchunk = -(-nseg // _CH)

    ins = []
    if not ones:
        ins.append(jax.ShapeDtypeStruct((2 * r, HH), jnp.float32))
    ins.append(jax.ShapeDtypeStruct((r,), jnp.int32))

    scratch = [
        pltpu.VMEM((sc,), jnp.int32),
        pltpu.VMEM((cap // g2, g2), jnp.int32),
        pltpu.VMEM((cap // g2, g2), jnp.int32),
        pltpu.VMEM((4, g2, HH), jnp.float32),
        pltpu.VMEM((128, HH), jnp.float32),
        pltpu.VMEM_SHARED((_CH + _NS, HH), jnp.float32),
        pltpu.SemaphoreType.DMA,
    ]

    @functools.partial(
        pl.kernel,
        out_type=jax.ShapeDtypeStruct((2, _rpad(nseg), HH), jnp.float32),
        mesh=_sc_mesh(),
        compiler_params=pltpu.CompilerParams(use_tc_tiling_on_sc=False,
                                             needs_layout_passes=False),
        scratch_types=scratch,
    )
    def k(*refs):
        pos = 0
        if not ones:
            vals_hbm = refs[pos]
            pos += 1
        idx_hbm, out_hbm, ib_v, idbuf_v, offbuf_v, rows_v, zb_v, acc, sem = (
            refs[pos:pos + 9])

        cid = lax.axis_index("c")
        sid = lax.axis_index("s")
        lane = lax.broadcasted_iota(jnp.int32, (_L,), 0)
        _fill_vmem(zb_v, 128, 0.0)
        if ones:
            _fill_vmem(rows_v.at[0], g2, 1.0)

        nt = (nstream - 1 - sid) // _NS + 1

        def chunk(b, _):
            base_off = b * _CH

            def zloop(j, _):
                pltpu.sync_copy(zb_v, acc.at[pl.ds(sid * per_t + j * 128, 128)])
                return 0

            lax.fori_loop(0, nzc, zloop, 0)
            pltpu.sync_copy(zb_v.at[pl.ds(0, 1)], acc.at[pl.ds(_CH + sid, 1)])
            plsc.subcore_barrier()

            def stream(t, cnt):
                gbase = (sid + t * _NS) * sc
                pltpu.sync_copy(idx_hbm.at[pl.ds(gbase, sc)], ib_v)

                def vloop(j, cnt):
                    dstv = ib_v[pl.ds(j * _L, _L)]
                    off = dstv - base_off
                    m = (off >= 0) & (off < _CH)
                    m32 = m.astype(jnp.int32)
                    p = cnt + plsc.cumsum(m32) - 1
                    m = m & (p < cap)
                    m32 = m.astype(jnp.int32)
                    gid = gbase + j * _L + lane + cid * r
                    plsc.store_scatter(idbuf_v, [lax.shift_right_logical(p, 8),
                                                 p & 255], gid, mask=m)
                    plsc.store_scatter(offbuf_v, [lax.shift_right_logical(p, 8),
                                                  p & 255], off, mask=m)
                    return cnt + jnp.sum(m32)

                return lax.fori_loop(0, sc // _L, vloop, cnt)

            cnt = lax.fori_loop(0, nt, stream, jnp.int32(0))

            cntp = (cnt + (g2 - 1)) & ~(g2 - 1)

            def pad(j, _):
                p = cnt + j * _L + lane
                m = p < cntp
                plsc.store_scatter(idbuf_v, [lax.shift_right_logical(p, 8),
                                             p & 255], lane + cid * r, mask=m)
                plsc.store_scatter(offbuf_v, [lax.shift_right_logical(p, 8),
                                              p & 255],
                                   jnp.full((_L,), _CH, jnp.int32) + sid,
                                   mask=m)
                return 0

            lax.fori_loop(0, g2 // _L, pad, 0)

            def gs(kk, _):
                if not ones:
                    rv = rows_v.at[kk & 3]
                    pltpu.async_copy(
                        vals_hbm.at[idbuf_v.at[kk]], rv, sem).wait()
                else:
                    rv = rows_v.at[0]
                pltpu.sync_copy(rv, acc.at[offbuf_v.at[kk]], add=True)
                return 0

            lax.fori_loop(0, lax.shift_right_logical(cntp, 8), gs, 0)
            plsc.subcore_barrier()

            def dump(j, _):
                pltpu.sync_copy(
                    acc.at[pl.ds(sid * per_t + j * 128, 128)],
                    out_hbm.at[cid, pl.ds(base_off + sid * per_t + j * 128, 128)])
                return 0

            lax.fori_loop(0, nzc, dump, 0)
            plsc.subcore_barrier()
            return 0

        lax.fori_loop(0, nchunk, chunk, 0)

    return k


@functools.lru_cache(maxsize=None)
def _sc_scatter_big_cached(r, nseg, ones):
    return _sc_scatter_big_make(r, nseg, ones)


def _conv(p, x, src, dst, eterm, cnt, n, blk, blk_e, big, stacked=False):
    xa, xb = _proj(x, p['mW1'][:H], p['mb1'], p['mW1'][H:2 * H], blk)
    e = src.shape[0]
    h, ps = _sc_pass1_cached(n, e, 200)(xa, xb, eterm, src, dst)
    st = _stats(ps, p['mg'], p['mB'], e)
    s = _silu_h(h, st, blk_e)
    if big:
        S = _sc_scatter_big_cached(e, n, False)(s.reshape(2 * e, HH), dst)
    else:
        S = _sc_scatter_small_cached(e, n, False, 0)(s, dst)
    z, psz = _phaseA(x, S, cnt, p['uW'][:H], p['uW'][H:], p['mW2'], p['mb2'],
                     p['ub'], blk)
    stz = _stats(psz, p['ug'], p['uB'], n)
    return _phaseB(z, stz, x, blk, stacked=stacked)


def kernel(x_element, x_props, edge_index, edge_attr, batch, line_graph_x,
           line_graph_edge_index, line_graph_edge_attr, line_graph_batch_mapping,
           params):
    n = x_element.shape[0]
    ne = edge_attr.shape[0]
    ng = 256
    blk_n = 2000
    blk_e = 4000

    src = edge_index[0].astype(jnp.int32)
    dst = edge_index[1].astype(jnp.int32)
    lsrc = line_graph_edge_index[0].astype(jnp.int32)
    ldst = line_graph_edge_index[1].astype(jnp.int32)
    batch32 = batch.astype(jnp.int32)
    lgbm = line_graph_batch_mapping.astype(jnp.int32)

    cnt_dst = _sc_scatter_small_cached(ne, n, True, 0)(dst)[0, :n, 0]
    cnt_src = _sc_scatter_small_cached(ne, n, True, 0)(src)[0, :n, 0]
    cnt_ldst = _sc_scatter_big_cached(ne, ne, True)(ldst)[0, :ne, 0]
    cnt_batch = _sc_scatter_small_cached(n, ng, True, 0)(batch32)[0, :, 0]
    cnt_lbatch = _sc_scatter_small_cached(ne, ng, True, n)(lgbm, batch32)[0, :, 0]

    emb_pad = jnp.zeros((128, 32), jnp.float32).at[:100].set(params['emb'])
    wE = emb_pad @ params['node_W'][:32]
    x = _node_init(x_element, x_props, wE, params['node_W'][32:],
                   params['node_b'], blk_n)
    lx = _rows_mm(line_graph_x, params['line_W'], params['line_b'], blk_e)

    def eterms(p, ea, blk):
        return _rows_mm(ea, p['mW1'][2 * H:], jnp.zeros((H,), jnp.float32), blk)

    eye = jnp.eye(H, dtype=jnp.float32)
    zH = jnp.zeros((H,), jnp.float32)

    lx_st = None
    for i in range(3):
        p = params['atom'][i]
        et = eterms(p, edge_attr, blk_e)
        x = _conv(p, x, src, dst, et, cnt_dst, n, blk_n, blk_e, False)
        q = params['line'][i]
        lt = eterms(q, line_graph_edge_attr, blk_e)
        lx, lx_st = _conv(q, lx, lsrc, ldst, lt, cnt_ldst, ne, blk_e, blk_e,
                          True, stacked=True)
        Sa = _sc_scatter_small_cached(ne, n, False, 0)(lx_st, src)
        qb = params['b2a'][i]
        z, psz = _phaseA(x, Sa, cnt_src, qb['W'][:H], qb['W'][H:], eye, zH,
                         qb['b'], blk_n)
        stz = _stats(psz, qb['g'], qb['B'], n)
        x = _phaseB(z, stz, None, blk_n)
    for i in range(3, 4):
        p = params['atom'][i]
        et = eterms(p, edge_attr, blk_e)
        x = _conv(p, x, src, dst, et, cnt_dst, n, blk_n, blk_e, False)
    p = params['atom'][4]
    et = eterms(p, edge_attr, blk_e)
    x, x_st = _conv(p, x, src, dst, et, cnt_dst, n, blk_n, blk_e, False,
                    stacked=True)

    aS = _sc_scatter_small_cached(n, ng, False, 0)(x_st, batch32)
    lS = _sc_scatter_small_cached(ne, ng, False, n)(lx_st, lgbm, batch32)
    return _final(aS, lS, cnt_batch, cnt_lbatch,
                  params['out_W'][:H], params['out_W'][H:], params['out_b'], ng)
